# baseline (speedup 1.0000x reference)
import math
import functools

import jax
import jax.numpy as jnp
from jax.experimental import pallas as pl
from jax.experimental.pallas import tpu as pltpu

NTS = ["piece", "square", "global"]
NN = {"piece": 32768, "square": 65536, "global": 1024}
ETS = [("piece", "piece"), ("piece", "square"), ("square", "square"),
       ("piece", "global"), ("square", "global")]
H = 4
DK = 32
HID = 128
L = 4


def _post_body(out_ref, x_ref, aW_ref, ab_ref, g_ref, b_ref, y_ref):
    o = jnp.dot(out_ref[...], aW_ref[...], preferred_element_type=jnp.float32)
    o = o + ab_ref[...] + x_ref[...]
    o = 0.5 * o * (1.0 + jax.lax.erf(o / math.sqrt(2.0)))
    m = jnp.mean(o, axis=-1, keepdims=True)
    v = jnp.mean((o - m) ** 2, axis=-1, keepdims=True)
    y_ref[...] = (o - m) * jax.lax.rsqrt(v + 1e-5) * g_ref[...] + b_ref[...]


def _post_ln(out, x, aW, ab, g, b):
    n = x.shape[0]
    blk = 512 if n >= 512 else n
    grid = (n // blk,)
    bs_row = pl.BlockSpec((blk, HID), lambda i: (i, 0))
    bs_w = pl.BlockSpec((HID, HID), lambda i: (0, 0))
    bs_v = pl.BlockSpec((1, HID), lambda i: (0, 0))
    return pl.pallas_call(
        _post_body,
        grid=grid,
        in_specs=[bs_row, bs_row, bs_w, bs_v, bs_v, bs_v],
        out_specs=bs_row,
        out_shape=jax.ShapeDtypeStruct((n, HID), jnp.float32),
    )(out, x, aW, ab.reshape(1, HID), g.reshape(1, HID), b.reshape(1, HID))


def kernel(x_piece, x_square, x_global, ei_0, ei_1, ei_2, ei_3, ei_4, ew_pp, params):
    eis = [ei_0, ei_1, ei_2, ei_3, ei_4]
    x = {"piece": x_piece, "square": x_square, "global": x_global}
    x = {nt: x[nt] @ params["projW_" + nt] + params["projb_" + nt] for nt in NTS}
    for l in range(L):
        k = {nt: (x[nt] @ params[f"kW_{l}_{nt}"] + params[f"kb_{l}_{nt}"]).reshape(-1, H, DK) for nt in NTS}
        q = {nt: (x[nt] @ params[f"qW_{l}_{nt}"] + params[f"qb_{l}_{nt}"]).reshape(-1, H, DK) for nt in NTS}
        v = {nt: (x[nt] @ params[f"vW_{l}_{nt}"] + params[f"vb_{l}_{nt}"]).reshape(-1, H, DK) for nt in NTS}
        out = {nt: jnp.zeros((NN[nt], H, DK), jnp.float32) for nt in NTS}
        for i, (st, dt) in enumerate(ETS):
            src = eis[i][0]
            dst = eis[i][1]
            k_j = k[st][src]
            q_i = q[dt][dst]
            v_j = v[st][src]
            k_att = jnp.einsum("ehd,hdk->ehk", k_j, params[f"att_{l}_{i}"])
            alpha = jnp.sum(k_att * q_i, axis=-1) * params[f"pri_{l}_{i}"] / math.sqrt(DK)
            if i == 0:
                alpha = alpha * (1.0 + ew_pp[:, None])
            amax = jax.ops.segment_max(alpha, dst, num_segments=NN[dt])
            amax = jnp.where(jnp.isfinite(amax), amax, 0.0)
            e = jnp.exp(alpha - amax[dst])
            denom = jax.ops.segment_sum(e, dst, num_segments=NN[dt])
            a = e / (denom[dst] + 1e-16)
            v_msg = jnp.einsum("ehd,hdk->ehk", v_j, params[f"msg_{l}_{i}"])
            out[dt] = out[dt] + jax.ops.segment_sum(v_msg * a[:, :, None], dst, num_segments=NN[dt])
        newx = {}
        for nt in NTS:
            newx[nt] = _post_ln(out[nt].reshape(-1, HID), x[nt],
                                params[f"aW_{l}_{nt}"], params[f"ab_{l}_{nt}"],
                                params[f"lng_{l}_{nt}"], params[f"lnb_{l}_{nt}"])
        x = newx
    return (x["piece"], x["square"], x["global"])
